# Initial kernel scaffold; baseline (speedup 1.0000x reference)
import jax
import jax.numpy as jnp
from jax.experimental import pallas as pl


def kernel(edge_index, edge_type, h_index, t_index, r_index, query_table, W_ih, W_hh, b_ih, b_hh, Wl, bl, w_lin, b_lin):
    raise NotImplementedError("write your pallas kernel here")


# speedup vs baseline: 2.9304x; 2.9304x over previous
import functools

import jax
import jax.numpy as jnp
from jax import lax
from jax.experimental import pallas as pl
from jax.experimental.pallas import tpu as pltpu
from jax.experimental.pallas import tpu_sc as plsc

_NR = 12
_H = 128
_STEPS = 4
_N = 10000
_NP1 = _N + 1
_B = 64
_E = 640000
_EPS = 1e-10

_NC = 2
_NS = 16
_NW = _NC * _NS

_CHUNK = 512
_SUB = 128
_CPW = 40
_EPAD = _NW * _CPW * _CHUNK
_NPAD = 10112
_RPT = _NPAD // _NS
_DUMMY = _N + 8


def _lstm(query, W_ih, W_hh, b_ih, b_hh):
    T, B, H = query.shape
    h = jnp.zeros((B, H), dtype=query.dtype)
    c = jnp.zeros((B, H), dtype=query.dtype)
    outs = []
    for t in range(T):
        gates = query[t] @ W_ih.T + b_ih + h @ W_hh.T + b_hh
        i, f, g, o = jnp.split(gates, 4, axis=-1)
        i = jax.nn.sigmoid(i)
        f = jax.nn.sigmoid(f)
        g = jnp.tanh(g)
        o = jax.nn.sigmoid(o)
        c = f * c + i * g
        h = o * jnp.tanh(c)
        outs.append(h)
    return jnp.stack(outs, axis=0)


def _mp_body(ein_hbm, eout_hbm, etyp_hbm, inp_hbm, w_hbm, zeros_hbm, out_hbm,
             iidx, tidx, oidx, rows, wrows, acc_sh, sem_g, sem_s):
    c = lax.axis_index("c")
    s = lax.axis_index("s")
    wid = s * _NC + c

    @pl.when(s == 0)
    def _():
        pltpu.sync_copy(zeros_hbm, acc_sh)
    plsc.subcore_barrier()

    def chunk_body(g, carry):
        base = (wid * _CPW + g) * _CHUNK
        row0 = (wid * _CPW + g) * (_CHUNK // _SUB)
        pltpu.sync_copy(ein_hbm.at[pl.ds(base, _CHUNK)], iidx)
        pltpu.sync_copy(etyp_hbm.at[pl.ds(base, _CHUNK)], tidx)
        pltpu.sync_copy(eout_hbm.at[pl.ds(row0, _CHUNK // _SUB)], oidx)
        d1 = pltpu.async_copy(inp_hbm.at[iidx], rows, sem_g)
        d2 = pltpu.async_copy(w_hbm.at[tidx], wrows, sem_g)
        d1.wait()
        d2.wait()

        def mul_body(r, carry2):
            for cg in range(4):
                sl = pl.ds(cg * 16, 16)
                rows[r, sl] = rows[r, sl] * wrows[r, sl]
            return carry2

        lax.fori_loop(0, _CHUNK, mul_body, 0)

        descs = [
            pltpu.async_copy(rows.at[pl.ds(j * _SUB, _SUB)],
                             acc_sh.at[oidx.at[j]], sem_s, add=True)
            for j in range(_CHUNK // _SUB)
        ]
        for d in descs:
            d.wait()
        return carry

    lax.fori_loop(0, _CPW, chunk_body, 0)

    plsc.subcore_barrier()
    pltpu.sync_copy(acc_sh.at[pl.ds(s * _RPT, _RPT)],
                    out_hbm.at[pl.ds(c * _NPAD + s * _RPT, _RPT)])


@jax.jit
def _mp_step(ein, eout, etyp, inp, w, zeros_np):
    mesh = plsc.VectorSubcoreMesh(core_axis_name="c", subcore_axis_name="s")
    f = pl.kernel(
        _mp_body,
        out_type=jax.ShapeDtypeStruct((_NC * _NPAD, _B), jnp.float32),
        mesh=mesh,
        compiler_params=pltpu.CompilerParams(use_tc_tiling_on_sc=False),
        scratch_types=[
            pltpu.VMEM((_CHUNK,), jnp.int32),
            pltpu.VMEM((_CHUNK,), jnp.int32),
            pltpu.VMEM((_CHUNK // _SUB, _SUB), jnp.int32),
            pltpu.VMEM((_CHUNK, _B), jnp.float32),
            pltpu.VMEM((_CHUNK, _B), jnp.float32),
            pltpu.VMEM_SHARED((_NPAD, _B), jnp.float32),
            pltpu.SemaphoreType.DMA,
            pltpu.SemaphoreType.DMA,
        ],
    )
    return f(ein, eout, etyp, inp, w, zeros_np)


def kernel(edge_index, edge_type, h_index, t_index, r_index, query_table,
           W_ih, W_hh, b_ih, b_hh, Wl, bl, w_lin, b_lin):
    is_t_neg = jnp.all(h_index == h_index[:, :1], axis=-1, keepdims=True)
    nh = jnp.where(is_t_neg, h_index, t_index)
    nt = jnp.where(is_t_neg, t_index, h_index)
    nr = jnp.where(is_t_neg, r_index, r_index + _NR)
    hr = nh * _NR + nr
    hr_flat = hr.reshape(-1)
    hr_set, inv = jnp.unique(hr_flat, return_inverse=True, size=hr_flat.shape[0])
    inv = jnp.asarray(inv).reshape(hr.shape)
    h_set = hr_set // _NR
    r_set = hr_set % _NR

    end_index = jnp.full_like(r_set, _NR)
    q_index = jnp.stack([r_set] * (_STEPS - 1) + [end_index], axis=0)
    query = query_table[q_index]
    hidden = _lstm(query, W_ih, W_hh, b_ih, b_hh)

    atts = []
    weights = []
    for i in range(_STEPS):
        key_h = hidden[i]
        x = jnp.einsum('bd,tbd->bt', key_h, hidden[:i + 1])
        atts.append(jax.nn.softmax(x, axis=-1))
        weights.append(jax.nn.softmax(key_h @ Wl.T + bl, axis=-1).T)

    pad = _EPAD - _E
    node_in = edge_index[0].astype(jnp.int32)
    node_out = edge_index[1].astype(jnp.int32)
    ein = jnp.concatenate([node_in, jnp.full((pad,), _DUMMY, jnp.int32)])
    eout = jnp.concatenate([node_out, jnp.full((pad,), _DUMMY, jnp.int32)])
    eout = eout.reshape(_EPAD // _SUB, _SUB)
    etyp = jnp.concatenate([edge_type.astype(jnp.int32),
                            jnp.zeros((pad,), jnp.int32)])

    zeros_np = jnp.zeros((_NPAD, _B), jnp.float32)
    mem0 = jax.nn.one_hot(h_set, _NPAD, dtype=jnp.float32).T
    mems = [mem0]
    out_n = None
    for i in range(_STEPS):
        att = atts[i]
        inp = mems[0] * att[:, 0][None]
        for t in range(1, i + 1):
            inp = inp + mems[t] * att[:, t][None]
        parts = _mp_step(ein, eout, etyp, inp, weights[i], zeros_np)
        out = parts[:_NPAD] + parts[_NPAD:]
        out_n = out / jnp.clip(out.sum(axis=0, keepdims=True), _EPS)
        mems.append(out_n)

    score = out_n[nt, inv]
    score = score * w_lin[0, 0] + b_lin[0]
    return score
